# Initial kernel scaffold; baseline (speedup 1.0000x reference)
import jax
import jax.numpy as jnp
from jax.experimental import pallas as pl


def kernel(query_feat, query_coords, value_feat, W_so, b_so, W_aw, b_aw, W_v, b_v, W_o, b_o):
    raise NotImplementedError("write your pallas kernel here")


# speedup vs baseline: 15.0530x; 15.0530x over previous
import functools

import jax
import jax.numpy as jnp
from jax import lax
from jax.experimental import pallas as pl
from jax.experimental.pallas import tpu as pltpu
from jax.experimental.pallas import tpu_sc as plsc

N_PTS = 32768
D_MODEL = 256
N_HEADS = 8
N_POINTS = 4

_HIGH = jax.lax.Precision.HIGHEST


def _dot(a, b, precision=_HIGH):
    return jax.lax.dot_general(
        a, b, (((1,), (0,)), ((), ())),
        precision=precision, preferred_element_type=jnp.float32)


def _vidx_body(c_ref, vidx_ref, rc_ref):
    n = c_ref.shape[1]
    c = c_ref[...]
    c0 = c[0:1, :]
    c1 = c[1:2, :]
    c2 = c[2:3, :]
    min0 = jnp.min(c0)
    min1 = jnp.min(c1)
    min2 = jnp.min(c2)
    rc0 = jnp.max(c0) - min0
    rc1 = jnp.max(c1) - min1
    rc2 = jnp.max(c2) - min2
    rv0 = (rc0 // 8 + 1).astype(jnp.float32)
    rv1 = (rc1 // 8 + 1).astype(jnp.float32)
    iv0 = (c0 - min0).astype(jnp.float32) * 0.125
    iv1 = (c1 - min1).astype(jnp.float32) * 0.125
    iv2 = (c2 - min2).astype(jnp.float32) * 0.125
    idxf = iv0 * rv1 * rv0 + iv1 * rv0 + iv2
    vidx_ref[...] = jnp.clip(idxf.astype(jnp.int32), 0, n - 1)
    rc_ref[0] = rc0
    rc_ref[1] = rc1
    rc_ref[2] = rc2


def _main_body(rc_ref, qf_ref, g_ref, Wso_ref, bso_ref, Waw_ref, baw_ref,
               Wv_ref, bv_ref, Wo_ref, bo_ref, out_ref):
    f32 = jnp.float32
    H, P = N_HEADS, N_POINTS
    HP = H * P
    HP3 = HP * 3
    dh = D_MODEL // H
    qf = qf_ref[...]
    so = _dot(qf, Wso_ref[...]) + bso_ref[...]
    aw = _dot(qf, Waw_ref[...]) + baw_ref[...]
    rc0 = rc_ref[0].astype(f32)
    rc1 = rc_ref[1].astype(f32)
    rc2 = rc_ref[2].astype(f32)
    comp = jax.lax.broadcasted_iota(jnp.int32, (1, HP3), 1) % 3
    half = jnp.where(comp == 0, rc0, jnp.where(comp == 1, rc1, rc2)) * 0.5
    soi = (so * half).astype(jnp.int32)
    mc = ((soi >= 0) & (soi < 8)).astype(f32)
    j3 = jax.lax.broadcasted_iota(jnp.int32, (HP3, HP), 0)
    g3 = jax.lax.broadcasted_iota(jnp.int32, (HP3, HP), 1)
    G3 = (j3 // 3 == g3).astype(f32)
    maskf = (_dot(mc, G3) > 2.5).astype(f32)
    e = jnp.exp(aw)
    ia = jax.lax.broadcasted_iota(jnp.int32, (HP, HP), 0)
    ja = jax.lax.broadcasted_iota(jnp.int32, (HP, HP), 1)
    A4 = (ia // P == ja // P).astype(f32)
    S = _dot(e, A4)
    wm = maskf * e / S
    jr = jax.lax.broadcasted_iota(jnp.int32, (HP, D_MODEL), 0)
    cr = jax.lax.broadcasted_iota(jnp.int32, (HP, D_MODEL), 1)
    Rm = (cr // dh == jr // P).astype(f32)
    w_rep = _dot(wm, Rm)
    v = _dot(g_ref[...], Wv_ref[...]) + bv_ref[...]
    out_ref[...] = _dot(v * w_rep, Wo_ref[...]) + bo_ref[...]


def _sc_gather(vf, idx3):
    n, d = vf.shape
    nw, nch, ch = idx3.shape
    bpw = nch * ch
    mesh = plsc.VectorSubcoreMesh(core_axis_name="c", subcore_axis_name="s")

    @functools.partial(
        pl.kernel, mesh=mesh,
        out_type=jax.ShapeDtypeStruct((n, d), jnp.float32),
        scratch_types=[
            pltpu.VMEM((nch, ch), jnp.int32),
            pltpu.VMEM((ch, d), jnp.float32),
            pltpu.VMEM((ch, d), jnp.float32),
            pltpu.SemaphoreType.DMA,
            pltpu.SemaphoreType.DMA,
        ],
    )
    def k(vf_hbm, idx_hbm, out_hbm, idx_v, buf0, buf1, sem0, sem1):
        wid = lax.axis_index("s") * 2 + lax.axis_index("c")
        base = wid * bpw
        pltpu.sync_copy(idx_hbm.at[wid], idx_v)
        bufs = (buf0, buf1)
        sems = (sem0, sem1)
        copies = [None, None]
        copies[0] = pltpu.async_copy(vf_hbm.at[idx_v.at[0]], bufs[0], sems[0])
        for ci in range(nch):
            b = ci % 2
            if ci + 1 < nch:
                nb = (ci + 1) % 2
                copies[nb] = pltpu.async_copy(
                    vf_hbm.at[idx_v.at[ci + 1]], bufs[nb], sems[nb])
            copies[b].wait()
            pltpu.sync_copy(bufs[b], out_hbm.at[pl.ds(base + ci * ch, ch)])

    return k(vf, idx3)


def kernel(query_feat, query_coords, value_feat, W_so, b_so, W_aw, b_aw,
           W_v, b_v, W_o, b_o):
    n, d = query_feat.shape
    coords_t = query_coords.astype(jnp.int32).T

    vidx2d, rc = pl.pallas_call(
        _vidx_body,
        out_shape=(
            jax.ShapeDtypeStruct((1, n), jnp.int32),
            jax.ShapeDtypeStruct((3,), jnp.int32),
        ),
        in_specs=[pl.BlockSpec((3, n), lambda: (0, 0))],
        out_specs=(
            pl.BlockSpec((1, n), lambda: (0, 0)),
            pl.BlockSpec(memory_space=pltpu.SMEM),
        ),
    )(coords_t)

    idx3 = vidx2d.reshape(32, n // 32 // 128, 128)
    g = _sc_gather(value_feat, idx3)

    BN = 2048
    grid = (n // BN,)
    full = lambda shape: pl.BlockSpec(shape, lambda i: (0, 0))
    out = pl.pallas_call(
        _main_body,
        grid=grid,
        in_specs=[
            pl.BlockSpec(memory_space=pltpu.SMEM),
            pl.BlockSpec((BN, d), lambda i: (i, 0)),
            pl.BlockSpec((BN, d), lambda i: (i, 0)),
            full((d, N_HEADS * N_POINTS * 3)),
            full((1, N_HEADS * N_POINTS * 3)),
            full((d, N_HEADS * N_POINTS)),
            full((1, N_HEADS * N_POINTS)),
            full((d, d)),
            full((1, d)),
            full((d, d)),
            full((1, d)),
        ],
        out_specs=pl.BlockSpec((BN, d), lambda i: (i, 0)),
        out_shape=jax.ShapeDtypeStruct((n, d), jnp.float32),
    )(rc, query_feat, g, W_so, b_so.reshape(1, -1), W_aw,
      b_aw.reshape(1, -1), W_v, b_v.reshape(1, -1), W_o, b_o.reshape(1, -1))
    return out
